# baseline (speedup 1.0000x reference)
import functools

import jax
import jax.numpy as jnp
from jax.experimental import pallas as pl
from jax.experimental.pallas import tpu as pltpu


def _se_fused_kernel(inv_s, x_ref, w1t_ref, b1_ref, w2t_ref, b2_ref, o_ref):
    x = x_ref[...]
    ones_row = jnp.ones((x.shape[0], x.shape[1]), jnp.float32)
    z = jax.lax.dot_general(
        ones_row, x, (((1,), (1,)), ((0,), (0,))),
        preferred_element_type=jnp.float32) * inv_s
    h = jnp.dot(z, w1t_ref[...], preferred_element_type=jnp.float32) + b1_ref[...]
    h = jnp.maximum(h, 0.0)
    g = jnp.dot(h, w2t_ref[...], preferred_element_type=jnp.float32) + b2_ref[...]
    g = jax.nn.sigmoid(g)
    o_ref[...] = (x * g[:, None, :]).astype(o_ref.dtype)


def kernel(x, w1, b1, w2, b2):
    B, C, D, H, W = x.shape
    Cr = w1.shape[0]
    S = D * H * W
    bB = 2 if B % 2 == 0 else 1

    xt = x.reshape(B, C, S).transpose(0, 2, 1)
    out = pl.pallas_call(
        functools.partial(_se_fused_kernel, 1.0 / float(S)),
        out_shape=jax.ShapeDtypeStruct((B, S, C), x.dtype),
        grid=(B // bB,),
        in_specs=[
            pl.BlockSpec((bB, S, C), lambda b: (b, 0, 0)),
            pl.BlockSpec((C, Cr), lambda b: (0, 0)),
            pl.BlockSpec((1, Cr), lambda b: (0, 0)),
            pl.BlockSpec((Cr, C), lambda b: (0, 0)),
            pl.BlockSpec((1, C), lambda b: (0, 0)),
        ],
        out_specs=pl.BlockSpec((bB, S, C), lambda b: (b, 0, 0)),
        compiler_params=pltpu.CompilerParams(
            dimension_semantics=("parallel",)),
    )(xt, w1.T, b1.reshape(1, Cr), w2.T, b2.reshape(1, C))
    return out.transpose(0, 2, 1).reshape(B, C, D, H, W)


# speedup vs baseline: 1.0035x; 1.0035x over previous
import functools

import jax
import jax.numpy as jnp
from jax.experimental import pallas as pl
from jax.experimental.pallas import tpu as pltpu


def _se_fused_kernel(inv_s, x_ref, w1t_ref, b1_ref, w2t_ref, b2_ref, o_ref):
    x = x_ref[...]
    z = jnp.sum(x, axis=1) * inv_s
    h = jnp.dot(z, w1t_ref[...], preferred_element_type=jnp.float32) + b1_ref[...]
    h = jnp.maximum(h, 0.0)
    g = jnp.dot(h, w2t_ref[...], preferred_element_type=jnp.float32) + b2_ref[...]
    g = jax.nn.sigmoid(g)
    o_ref[...] = (x * g[:, None, :]).astype(o_ref.dtype)


def kernel(x, w1, b1, w2, b2):
    B, C, D, H, W = x.shape
    Cr = w1.shape[0]
    S = D * H * W
    bB = 2 if B % 2 == 0 else 1

    xt = x.reshape(B, C, S).transpose(0, 2, 1)
    out = pl.pallas_call(
        functools.partial(_se_fused_kernel, 1.0 / float(S)),
        out_shape=jax.ShapeDtypeStruct((B, S, C), x.dtype),
        grid=(B // bB,),
        in_specs=[
            pl.BlockSpec((bB, S, C), lambda b: (b, 0, 0)),
            pl.BlockSpec((C, Cr), lambda b: (0, 0)),
            pl.BlockSpec((1, Cr), lambda b: (0, 0)),
            pl.BlockSpec((Cr, C), lambda b: (0, 0)),
            pl.BlockSpec((1, C), lambda b: (0, 0)),
        ],
        out_specs=pl.BlockSpec((bB, S, C), lambda b: (b, 0, 0)),
        compiler_params=pltpu.CompilerParams(
            dimension_semantics=("parallel",)),
    )(xt, w1.T, b1.reshape(1, Cr), w2.T, b2.reshape(1, C))
    return out.transpose(0, 2, 1).reshape(B, C, D, H, W)
